# Initial kernel scaffold; baseline (speedup 1.0000x reference)
import jax
import jax.numpy as jnp
from jax.experimental import pallas as pl


def kernel(x, edge_index, Wl1, Wr1, b1, Wl2, Wr2, b2, Wc1, bc1, Wc2, bc2, Wc3, bc3):
    raise NotImplementedError("write your pallas kernel here")


# speedup vs baseline: 5.5359x; 5.5359x over previous
import jax
import jax.numpy as jnp
from jax import lax
from jax.experimental import pallas as pl
from jax.experimental.pallas import tpu as pltpu
from jax.experimental.pallas import tpu_sc as plsc

N = 10000
D = 128
E = 320000

NC = 2
NS = 16
NW = NC * NS
EPW = E // NW
K = 80
CHUNKS = EPW // K
ZR = 80
ZCHUNKS = N // ZR
ZPT = -(-ZCHUNKS // NS)
L = 16


def _make_sc_agg(with_counts):
  mesh = plsc.VectorSubcoreMesh(core_axis_name="c", subcore_axis_name="s")
  out_type = [jax.ShapeDtypeStruct((NC, N, D), jnp.float32)]
  if with_counts:
    out_type.append(jax.ShapeDtypeStruct((NW, N), jnp.float32))
  scratch = [
      pltpu.VMEM_SHARED((N, D), jnp.float32),
      pltpu.VMEM((K,), jnp.int32),
      pltpu.VMEM((K,), jnp.int32),
      pltpu.VMEM((K, D), jnp.float32),
      pltpu.SemaphoreType.DMA,
  ]
  if with_counts:
    scratch.append(pltpu.VMEM((N,), jnp.float32))

  def body(h_hbm, src_hbm, dst_hbm, z_hbm, *outs_and_scratch):
    if with_counts:
      (part_hbm, hist_hbm, acc, srcv, dstv, rows, sem,
       hist) = outs_and_scratch
    else:
      (part_hbm, acc, srcv, dstv, rows, sem) = outs_and_scratch
      hist_hbm = hist = None
    stage = rows
    c = lax.axis_index("c")
    s = lax.axis_index("s")
    wid = c * NS + s

    pltpu.sync_copy(z_hbm, stage)
    for t in range(ZPT):
      j = s + NS * t

      @pl.when(j < ZCHUNKS)
      def _():
        pltpu.sync_copy(stage, acc.at[pl.ds(j * ZR, ZR), :])

    if with_counts:
      zeros16 = jnp.zeros((L,), jnp.float32)

      def zstep(i, carry):
        hist[pl.ds(i * L, L)] = zeros16
        return carry

      lax.fori_loop(0, N // L, zstep, 0)
    plsc.subcore_barrier()

    base_w = wid * EPW

    def step(i, carry):
      base = base_w + i * K
      pltpu.sync_copy(src_hbm.at[pl.ds(base, K)], srcv)
      pltpu.sync_copy(dst_hbm.at[pl.ds(base, K)], dstv)
      pltpu.async_copy(h_hbm.at[srcv], rows, sem).wait()
      pltpu.sync_copy(rows, acc.at[dstv], add=True)
      if with_counts:
        ones16 = jnp.ones((L,), jnp.float32)
        for j in range(K // L):
          idx = dstv[pl.ds(j * L, L)]
          plsc.addupdate_scatter(hist, [idx], ones16)
      return carry

    lax.fori_loop(0, CHUNKS, step, 0)
    plsc.subcore_barrier()

    for t in range(ZPT):
      j = s + NS * t

      @pl.when(j < ZCHUNKS)
      def _():
        rr = pl.ds(j * ZR, ZR)
        pltpu.sync_copy(acc.at[rr, :], stage)
        pltpu.sync_copy(stage, part_hbm.at[c, rr, :])

    if with_counts:
      pltpu.sync_copy(hist, hist_hbm.at[wid])

  return pl.kernel(body, out_type=tuple(out_type), mesh=mesh,
                   scratch_types=tuple(scratch),
                   compiler_params=pltpu.CompilerParams(
                       needs_layout_passes=False))


_sc_agg_counts = _make_sc_agg(True)
_sc_agg = _make_sc_agg(False)


_BN = 1000


def _mean(p_ref, c_ref):
  agg = p_ref[0] + p_ref[1]
  cnt = jnp.sum(c_ref[...], axis=1)
  return agg / jnp.maximum(cnt, 1.0)[:, None]


def _tc1_body(p_ref, c_ref, x_ref, wl_ref, wr_ref, b_ref, o_ref):
  mean = _mean(p_ref, c_ref)
  out = (jnp.dot(mean, wl_ref[...], preferred_element_type=jnp.float32)
         + jnp.dot(x_ref[...], wr_ref[...], preferred_element_type=jnp.float32)
         + b_ref[...])
  o_ref[...] = jnp.maximum(out, 0.0)


def _tc1(p, c, x, wl, wr, b):
  return pl.pallas_call(
      _tc1_body,
      grid=(N // _BN,),
      in_specs=[
          pl.BlockSpec((NC, _BN, D), lambda i: (0, i, 0)),
          pl.BlockSpec((_BN, NW), lambda i: (i, 0)),
          pl.BlockSpec((_BN, D), lambda i: (i, 0)),
          pl.BlockSpec((D, D), lambda i: (0, 0)),
          pl.BlockSpec((D, D), lambda i: (0, 0)),
          pl.BlockSpec((D,), lambda i: (0,)),
      ],
      out_specs=pl.BlockSpec((_BN, D), lambda i: (i, 0)),
      out_shape=jax.ShapeDtypeStruct((N, D), jnp.float32),
  )(p, c, x, wl, wr, b)


def _tc2_body(p_ref, c_ref, h_ref, wl_ref, wr_ref, b_ref,
              wc1_ref, bc1_ref, wc2_ref, bc2_ref, wc3_ref, bc3_ref,
              emb_ref, prob_ref):
  mean = _mean(p_ref, c_ref)
  emb = (jnp.dot(mean, wl_ref[...], preferred_element_type=jnp.float32)
         + jnp.dot(h_ref[...], wr_ref[...], preferred_element_type=jnp.float32)
         + b_ref[...])
  emb_ref[...] = emb
  t = jnp.maximum(
      jnp.dot(emb, wc1_ref[...], preferred_element_type=jnp.float32)
      + bc1_ref[...], 0.0)
  t = jnp.maximum(
      jnp.dot(t, wc2_ref[...], preferred_element_type=jnp.float32)
      + bc2_ref[...], 0.0)
  logit = jnp.dot(t, wc3_ref[...], preferred_element_type=jnp.float32) \
      + bc3_ref[...]
  prob_ref[...] = jax.nn.sigmoid(logit)


def _tc2(p, c, h, wl, wr, b, wc1, bc1, wc2, bc2, wc3, bc3):
  return pl.pallas_call(
      _tc2_body,
      grid=(N // _BN,),
      in_specs=[
          pl.BlockSpec((NC, _BN, D), lambda i: (0, i, 0)),
          pl.BlockSpec((_BN, NW), lambda i: (i, 0)),
          pl.BlockSpec((_BN, D), lambda i: (i, 0)),
          pl.BlockSpec((D, D), lambda i: (0, 0)),
          pl.BlockSpec((D, D), lambda i: (0, 0)),
          pl.BlockSpec((D,), lambda i: (0,)),
          pl.BlockSpec((D, D), lambda i: (0, 0)),
          pl.BlockSpec((D,), lambda i: (0,)),
          pl.BlockSpec((D, D // 2), lambda i: (0, 0)),
          pl.BlockSpec((D // 2,), lambda i: (0,)),
          pl.BlockSpec((D // 2, 1), lambda i: (0, 0)),
          pl.BlockSpec((1,), lambda i: (0,)),
      ],
      out_specs=[
          pl.BlockSpec((_BN, D), lambda i: (i, 0)),
          pl.BlockSpec((_BN, 1), lambda i: (i, 0)),
      ],
      out_shape=[
          jax.ShapeDtypeStruct((N, D), jnp.float32),
          jax.ShapeDtypeStruct((N, 1), jnp.float32),
      ],
  )(p, c, h, wl, wr, b, wc1, bc1, wc2, bc2, wc3, bc3)


def kernel(x, edge_index, Wl1, Wr1, b1, Wl2, Wr2, b2,
           Wc1, bc1, Wc2, bc2, Wc3, bc3):
  src = edge_index[0]
  dst = edge_index[1]
  z = jnp.zeros((ZR, D), jnp.float32)

  p1, cnt = _sc_agg_counts(x, src, dst, z)
  cnt = cnt.T
  h = _tc1(p1, cnt, x, Wl1, Wr1, b1)
  (p2,) = _sc_agg(h, src, dst, z)
  emb, probs = _tc2(p2, cnt, h, Wl2, Wr2, b2, Wc1, bc1, Wc2, bc2, Wc3, bc3)
  return (emb, probs)


# speedup vs baseline: 14.1083x; 2.5485x over previous
import jax
import jax.numpy as jnp
from jax import lax
from jax.experimental import pallas as pl
from jax.experimental.pallas import tpu as pltpu
from jax.experimental.pallas import tpu_sc as plsc

N = 10000
D = 128
E = 320000

NC = 2
NS = 16
NW = NC * NS
EPW = E // NW
K = 80
CHUNKS = EPW // K
NBUF = 3
GROUPS = CHUNKS // NBUF
TAIL = CHUNKS - GROUPS * NBUF
GK = NBUF * K
ZR = 80
ZCHUNKS = N // ZR
ZPT = -(-ZCHUNKS // NS)
L = 16


def _make_sc_agg(with_counts):
  mesh = plsc.VectorSubcoreMesh(core_axis_name="c", subcore_axis_name="s")
  out_type = [jax.ShapeDtypeStruct((NC, N, D), jnp.float32)]
  if with_counts:
    out_type.append(jax.ShapeDtypeStruct((NW, N), jnp.float32))
  scratch = [
      pltpu.VMEM_SHARED((N, D), jnp.float32),
      [pltpu.VMEM((GK,), jnp.int32) for _ in range(2)],
      [pltpu.VMEM((GK,), jnp.int32) for _ in range(2)],
      [pltpu.VMEM((K,), jnp.int32) for _ in range(NBUF)],
      [pltpu.VMEM((K, D), jnp.float32) for _ in range(NBUF)],
      [pltpu.SemaphoreType.DMA for _ in range(NBUF)],
      [pltpu.SemaphoreType.DMA for _ in range(NBUF)],
      pltpu.SemaphoreType.DMA,
  ]
  if with_counts:
    scratch.append(pltpu.VMEM((N,), jnp.float32))

  def body(h_hbm, src_hbm, dst_hbm, z_hbm, *outs_and_scratch):
    if with_counts:
      (part_hbm, hist_hbm, acc, sgrp, dgrp, dstv, rows, gsem, ssem, isem,
       hist) = outs_and_scratch
    else:
      (part_hbm, acc, sgrp, dgrp, dstv, rows, gsem, ssem,
       isem) = outs_and_scratch
      hist_hbm = hist = None
    c = lax.axis_index("c")
    s = lax.axis_index("s")
    wid = c * NS + s
    base_w = wid * EPW
    ones16 = jnp.ones((L,), jnp.float32)

    pltpu.sync_copy(z_hbm, rows[0])
    for t in range(ZPT):
      j = s + NS * t
      if t * NS + NS <= ZCHUNKS:
        pltpu.async_copy(rows[0], acc.at[pl.ds(j * ZR, ZR), :], gsem[0])
      else:
        @pl.when(j < ZCHUNKS)
        def _():
          pltpu.async_copy(rows[0], acc.at[pl.ds(j * ZR, ZR), :], gsem[0])
    if with_counts:
      zeros16 = jnp.zeros((L,), jnp.float32)

      def zstep(i, carry):
        hist[pl.ds(i * L, L)] = zeros16
        return carry

      lax.fori_loop(0, N // L, zstep, 0)
    for t in range(ZPT):
      if t * NS + NS <= ZCHUNKS:
        pltpu.make_async_copy(rows[0], acc.at[pl.ds(0, ZR), :],
                              gsem[0]).wait()
      else:
        @pl.when(s + NS * t < ZCHUNKS)
        def _():
          pltpu.make_async_copy(rows[0], acc.at[pl.ds(0, ZR), :],
                                gsem[0]).wait()
    plsc.subcore_barrier()

    def copy_dstv(g2ref, b, off):
      for j in range(K // L):
        dstv[b][pl.ds(j * L, L)] = g2ref[pl.ds(off + j * L, L)]

    def hist_update(b):
      if with_counts:
        for j in range(K // L):
          idx = dstv[b][pl.ds(j * L, L)]
          plsc.addupdate_scatter(hist, [idx], ones16)

    def fire_gather(sref, off, b):
      pltpu.async_copy(h_hbm.at[sref.at[pl.ds(off, K)]], rows[b], gsem[b])

    def fire_scatter(b):
      pltpu.async_copy(rows[b], acc.at[dstv[b]], ssem[b], add=True)

    def wait_gather(b):
      pltpu.make_async_copy(h_hbm.at[dstv[b]], rows[b], gsem[b]).wait()

    def wait_scatter(b):
      pltpu.make_async_copy(rows[b], acc.at[dstv[b]], ssem[b]).wait()

    def prefetch_idx(g2, base):
      pltpu.async_copy(src_hbm.at[pl.ds(base, GK)], sgrp[g2], isem)
      pltpu.async_copy(dst_hbm.at[pl.ds(base, GK)], dgrp[g2], isem)

    def wait_idx(g2):
      pltpu.make_async_copy(src_hbm.at[pl.ds(0, GK)], sgrp[g2], isem).wait()
      pltpu.make_async_copy(dst_hbm.at[pl.ds(0, GK)], dgrp[g2], isem).wait()

    prefetch_idx(0, base_w)
    wait_idx(0)

    def group(g, g2):
      for b in range(NBUF):
        if b == 0:
          @pl.when(g > 0)
          def _():
            wait_idx(g2)

        @pl.when(g > 0)
        def _():
          wait_scatter(b)

        copy_dstv(dgrp[g2], b, b * K)
        fire_gather(sgrp[g2], b * K, b)

        if b == 0:
          @pl.when(g > 0)
          def _():
            bb = NBUF - 1
            wait_gather(bb)
            fire_scatter(bb)
            hist_update(bb)

          @pl.when(g < GROUPS - 1)
          def _():
            prefetch_idx(1 - g2, base_w + (g + 1) * GK)
        else:
          wait_gather(b - 1)
          fire_scatter(b - 1)
          hist_update(b - 1)

    def double_group(gg, carry):
      group(2 * gg, 0)
      group(2 * gg + 1, 1)
      return carry

    lax.fori_loop(0, GROUPS // 2, double_group, 0)
    if GROUPS % 2:
      group(jnp.int32(GROUPS - 1), (GROUPS - 1) % 2)

    wait_gather(NBUF - 1)
    fire_scatter(NBUF - 1)
    hist_update(NBUF - 1)
    for t in range(TAIL):
      i = GROUPS * NBUF + t
      b = t % NBUF
      wait_scatter(b)
      pltpu.sync_copy(dst_hbm.at[pl.ds(base_w + i * K, K)], dstv[b])
      pltpu.sync_copy(src_hbm.at[pl.ds(base_w + i * K, K)],
                      sgrp[0].at[pl.ds(0, K)])
      pltpu.async_copy(h_hbm.at[sgrp[0].at[pl.ds(0, K)]], rows[b], gsem[b])
      wait_gather(b)
      fire_scatter(b)
      hist_update(b)
    for b in range(NBUF):
      wait_scatter(b)
    plsc.subcore_barrier()

    def wait_write(b):
      pltpu.make_async_copy(rows[b], part_hbm.at[c, pl.ds(0, ZR), :],
                            ssem[b]).wait()

    for t in range(ZPT):
      j = s + NS * t
      b = t % NBUF

      @pl.when(j < ZCHUNKS)
      def _():
        rr = pl.ds(j * ZR, ZR)
        if t >= NBUF:
          wait_write(b)
        pltpu.sync_copy(acc.at[rr, :], rows[b])
        pltpu.async_copy(rows[b], part_hbm.at[c, rr, :], ssem[b])

    for b in range(NBUF):
      wait_write(b)

    if with_counts:
      pltpu.sync_copy(hist, hist_hbm.at[wid])

  return pl.kernel(body, out_type=tuple(out_type), mesh=mesh,
                   scratch_types=tuple(scratch),
                   compiler_params=pltpu.CompilerParams(
                       needs_layout_passes=False))


_sc_agg_counts = _make_sc_agg(True)
_sc_agg = _make_sc_agg(False)


_BN = 1000


def _mean(p_ref, c_ref):
  agg = p_ref[0] + p_ref[1]
  cnt = jnp.sum(c_ref[...], axis=1)
  return agg / jnp.maximum(cnt, 1.0)[:, None]


def _tc1_body(p_ref, c_ref, x_ref, wl_ref, wr_ref, b_ref, o_ref):
  mean = _mean(p_ref, c_ref)
  out = (jnp.dot(mean, wl_ref[...], preferred_element_type=jnp.float32)
         + jnp.dot(x_ref[...], wr_ref[...], preferred_element_type=jnp.float32)
         + b_ref[...])
  o_ref[...] = jnp.maximum(out, 0.0)


def _tc1(p, c, x, wl, wr, b):
  return pl.pallas_call(
      _tc1_body,
      grid=(N // _BN,),
      in_specs=[
          pl.BlockSpec((NC, _BN, D), lambda i: (0, i, 0)),
          pl.BlockSpec((_BN, NW), lambda i: (i, 0)),
          pl.BlockSpec((_BN, D), lambda i: (i, 0)),
          pl.BlockSpec((D, D), lambda i: (0, 0)),
          pl.BlockSpec((D, D), lambda i: (0, 0)),
          pl.BlockSpec((D,), lambda i: (0,)),
      ],
      out_specs=pl.BlockSpec((_BN, D), lambda i: (i, 0)),
      out_shape=jax.ShapeDtypeStruct((N, D), jnp.float32),
  )(p, c, x, wl, wr, b)


def _tc2_body(p_ref, c_ref, h_ref, wl_ref, wr_ref, b_ref,
              wc1_ref, bc1_ref, wc2_ref, bc2_ref, wc3_ref, bc3_ref,
              emb_ref, prob_ref):
  mean = _mean(p_ref, c_ref)
  emb = (jnp.dot(mean, wl_ref[...], preferred_element_type=jnp.float32)
         + jnp.dot(h_ref[...], wr_ref[...], preferred_element_type=jnp.float32)
         + b_ref[...])
  emb_ref[...] = emb
  t = jnp.maximum(
      jnp.dot(emb, wc1_ref[...], preferred_element_type=jnp.float32)
      + bc1_ref[...], 0.0)
  t = jnp.maximum(
      jnp.dot(t, wc2_ref[...], preferred_element_type=jnp.float32)
      + bc2_ref[...], 0.0)
  logit = jnp.dot(t, wc3_ref[...], preferred_element_type=jnp.float32) \
      + bc3_ref[...]
  prob_ref[...] = jax.nn.sigmoid(logit)


def _tc2(p, c, h, wl, wr, b, wc1, bc1, wc2, bc2, wc3, bc3):
  return pl.pallas_call(
      _tc2_body,
      grid=(N // _BN,),
      in_specs=[
          pl.BlockSpec((NC, _BN, D), lambda i: (0, i, 0)),
          pl.BlockSpec((_BN, NW), lambda i: (i, 0)),
          pl.BlockSpec((_BN, D), lambda i: (i, 0)),
          pl.BlockSpec((D, D), lambda i: (0, 0)),
          pl.BlockSpec((D, D), lambda i: (0, 0)),
          pl.BlockSpec((D,), lambda i: (0,)),
          pl.BlockSpec((D, D), lambda i: (0, 0)),
          pl.BlockSpec((D,), lambda i: (0,)),
          pl.BlockSpec((D, D // 2), lambda i: (0, 0)),
          pl.BlockSpec((D // 2,), lambda i: (0,)),
          pl.BlockSpec((D // 2, 1), lambda i: (0, 0)),
          pl.BlockSpec((1,), lambda i: (0,)),
      ],
      out_specs=[
          pl.BlockSpec((_BN, D), lambda i: (i, 0)),
          pl.BlockSpec((_BN, 1), lambda i: (i, 0)),
      ],
      out_shape=[
          jax.ShapeDtypeStruct((N, D), jnp.float32),
          jax.ShapeDtypeStruct((N, 1), jnp.float32),
      ],
  )(p, c, h, wl, wr, b, wc1, bc1, wc2, bc2, wc3, bc3)


def kernel(x, edge_index, Wl1, Wr1, b1, Wl2, Wr2, b2,
           Wc1, bc1, Wc2, bc2, Wc3, bc3):
  src = edge_index[0]
  dst = edge_index[1]
  z = jnp.zeros((ZR, D), jnp.float32)

  p1, cnt = _sc_agg_counts(x, src, dst, z)
  cnt = cnt.T
  h = _tc1(p1, cnt, x, Wl1, Wr1, b1)
  (p2,) = _sc_agg(h, src, dst, z)
  emb, probs = _tc2(p2, cnt, h, Wl2, Wr2, b2, Wc1, bc1, Wc2, bc2, Wc3, bc3)
  return (emb, probs)


# speedup vs baseline: 14.9551x; 1.0600x over previous
import jax
import jax.numpy as jnp
from jax import lax
from jax.experimental import pallas as pl
from jax.experimental.pallas import tpu as pltpu
from jax.experimental.pallas import tpu_sc as plsc

N = 10000
D = 128
E = 320000

NC = 2
NS = 16
NW = NC * NS
EPW = E // NW
K = 80
CHUNKS = EPW // K
NBUF = 3
GROUPS = CHUNKS // NBUF
TAIL = CHUNKS - GROUPS * NBUF
GK = NBUF * K
ZR = 80
ZCHUNKS = N // ZR
ZPT = -(-ZCHUNKS // NS)
L = 16


def _make_sc_agg(with_counts):
  mesh = plsc.VectorSubcoreMesh(core_axis_name="c", subcore_axis_name="s")
  out_type = [jax.ShapeDtypeStruct((NC, N, D), jnp.float32)]
  if with_counts:
    out_type.append(jax.ShapeDtypeStruct((NW, N), jnp.float32))
  scratch = [
      pltpu.VMEM_SHARED((N, D), jnp.float32),
      [pltpu.VMEM((GK,), jnp.int32) for _ in range(2)],
      [pltpu.VMEM((GK,), jnp.int32) for _ in range(2)],
      [pltpu.VMEM((K,), jnp.int32) for _ in range(NBUF)],
      [pltpu.VMEM((K, D), jnp.float32) for _ in range(NBUF)],
      [pltpu.SemaphoreType.DMA for _ in range(NBUF)],
      [pltpu.SemaphoreType.DMA for _ in range(NBUF)],
      pltpu.SemaphoreType.DMA,
  ]
  if with_counts:
    scratch.append(pltpu.VMEM((N,), jnp.float32))

  def body(h_hbm, src_hbm, dst_hbm, z_hbm, *outs_and_scratch):
    if with_counts:
      (part_hbm, hist_hbm, acc, sgrp, dgrp, dstv, rows, gsem, ssem, isem,
       hist) = outs_and_scratch
    else:
      (part_hbm, acc, sgrp, dgrp, dstv, rows, gsem, ssem,
       isem) = outs_and_scratch
      hist_hbm = hist = None
    c = lax.axis_index("c")
    s = lax.axis_index("s")
    wid = c * NS + s
    base_w = wid * EPW
    ones16 = jnp.ones((L,), jnp.float32)

    pltpu.sync_copy(z_hbm, rows[0])
    for t in range(ZPT):
      j = s + NS * t
      if t * NS + NS <= ZCHUNKS:
        pltpu.async_copy(rows[0], acc.at[pl.ds(j * ZR, ZR), :], gsem[0])
      else:
        @pl.when(j < ZCHUNKS)
        def _():
          pltpu.async_copy(rows[0], acc.at[pl.ds(j * ZR, ZR), :], gsem[0])
    if with_counts:
      zeros16 = jnp.zeros((L,), jnp.float32)

      def zstep(i, carry):
        hist[pl.ds(i * L, L)] = zeros16
        return carry

      lax.fori_loop(0, N // L, zstep, 0)
    for t in range(ZPT):
      if t * NS + NS <= ZCHUNKS:
        pltpu.make_async_copy(rows[0], acc.at[pl.ds(0, ZR), :],
                              gsem[0]).wait()
      else:
        @pl.when(s + NS * t < ZCHUNKS)
        def _():
          pltpu.make_async_copy(rows[0], acc.at[pl.ds(0, ZR), :],
                                gsem[0]).wait()
    plsc.subcore_barrier()

    def copy_dstv(g2ref, b, off):
      for j in range(K // L):
        dstv[b][pl.ds(j * L, L)] = g2ref[pl.ds(off + j * L, L)]

    def hist_update(b):
      if with_counts:
        for j in range(K // L):
          idx = dstv[b][pl.ds(j * L, L)]
          plsc.addupdate_scatter(hist, [idx], ones16)

    def fire_gather(sref, off, b):
      pltpu.async_copy(h_hbm.at[sref.at[pl.ds(off, K)]], rows[b], gsem[b])

    def fire_scatter(b):
      pltpu.async_copy(rows[b], acc.at[dstv[b]], ssem[b], add=True)

    def wait_gather(b):
      pltpu.make_async_copy(h_hbm.at[dstv[b]], rows[b], gsem[b]).wait()

    def wait_scatter(b):
      pltpu.make_async_copy(rows[b], acc.at[dstv[b]], ssem[b]).wait()

    def prefetch_idx(g2, base):
      pltpu.async_copy(src_hbm.at[pl.ds(base, GK)], sgrp[g2], isem)
      pltpu.async_copy(dst_hbm.at[pl.ds(base, GK)], dgrp[g2], isem)

    def wait_idx(g2):
      pltpu.make_async_copy(src_hbm.at[pl.ds(0, GK)], sgrp[g2], isem).wait()
      pltpu.make_async_copy(dst_hbm.at[pl.ds(0, GK)], dgrp[g2], isem).wait()

    prefetch_idx(0, base_w)
    wait_idx(0)

    def group(g, g2):
      for b in range(NBUF):
        if b == 0:
          @pl.when(g > 0)
          def _():
            wait_idx(g2)

        @pl.when(g > 0)
        def _():
          wait_scatter(b)

        copy_dstv(dgrp[g2], b, b * K)
        fire_gather(sgrp[g2], b * K, b)

        if b == 0:
          @pl.when(g > 0)
          def _():
            bb = NBUF - 1
            wait_gather(bb)
            fire_scatter(bb)
            hist_update(bb)

          @pl.when(g < GROUPS - 1)
          def _():
            prefetch_idx(1 - g2, base_w + (g + 1) * GK)
        else:
          wait_gather(b - 1)
          fire_scatter(b - 1)
          hist_update(b - 1)

    def double_group(gg, carry):
      group(2 * gg, 0)
      group(2 * gg + 1, 1)
      return carry

    lax.fori_loop(0, GROUPS // 2, double_group, 0)
    if GROUPS % 2:
      group(jnp.int32(GROUPS - 1), (GROUPS - 1) % 2)

    wait_gather(NBUF - 1)
    fire_scatter(NBUF - 1)
    hist_update(NBUF - 1)
    for t in range(TAIL):
      i = GROUPS * NBUF + t
      b = t % NBUF
      wait_scatter(b)
      pltpu.sync_copy(dst_hbm.at[pl.ds(base_w + i * K, K)], dstv[b])
      pltpu.sync_copy(src_hbm.at[pl.ds(base_w + i * K, K)],
                      sgrp[0].at[pl.ds(0, K)])
      pltpu.async_copy(h_hbm.at[sgrp[0].at[pl.ds(0, K)]], rows[b], gsem[b])
      wait_gather(b)
      fire_scatter(b)
      hist_update(b)
    for b in range(NBUF):
      wait_scatter(b)
    plsc.subcore_barrier()

    def wait_write(b):
      pltpu.make_async_copy(rows[b], part_hbm.at[c, pl.ds(0, ZR), :],
                            ssem[b]).wait()

    for t in range(ZPT):
      j = s + NS * t
      b = t % NBUF

      @pl.when(j < ZCHUNKS)
      def _():
        rr = pl.ds(j * ZR, ZR)
        if t >= NBUF:
          wait_write(b)
        pltpu.sync_copy(acc.at[rr, :], rows[b])
        pltpu.async_copy(rows[b], part_hbm.at[c, rr, :], ssem[b])

    for b in range(NBUF):
      wait_write(b)

    if with_counts:
      pltpu.sync_copy(hist, hist_hbm.at[wid])

  return pl.kernel(body, out_type=tuple(out_type), mesh=mesh,
                   scratch_types=tuple(scratch),
                   compiler_params=pltpu.CompilerParams(
                       needs_layout_passes=False))


_sc_agg_counts = _make_sc_agg(True)
_sc_agg = _make_sc_agg(False)


def _mean(p_ref, c_ref):
  agg = p_ref[0] + p_ref[1]
  cnt = jnp.sum(c_ref[...], axis=0)
  return agg / jnp.maximum(cnt, 1.0)[:, None]


def _tc1_body(p_ref, c_ref, x_ref, wl_ref, wr_ref, b_ref, o_ref):
  mean = _mean(p_ref, c_ref)
  out = (jnp.dot(mean, wl_ref[...], preferred_element_type=jnp.float32)
         + jnp.dot(x_ref[...], wr_ref[...], preferred_element_type=jnp.float32)
         + b_ref[...])
  o_ref[...] = jnp.maximum(out, 0.0)


def _tc1(p, c, x, wl, wr, b):
  return pl.pallas_call(
      _tc1_body,
      out_shape=jax.ShapeDtypeStruct((N, D), jnp.float32),
  )(p, c, x, wl, wr, b)


def _tc2_body(p_ref, c_ref, h_ref, wl_ref, wr_ref, b_ref,
              wc1_ref, bc1_ref, wc2_ref, bc2_ref, wc3_ref, bc3_ref,
              emb_ref, prob_ref):
  mean = _mean(p_ref, c_ref)
  emb = (jnp.dot(mean, wl_ref[...], preferred_element_type=jnp.float32)
         + jnp.dot(h_ref[...], wr_ref[...], preferred_element_type=jnp.float32)
         + b_ref[...])
  emb_ref[...] = emb
  t = jnp.maximum(
      jnp.dot(emb, wc1_ref[...], preferred_element_type=jnp.float32)
      + bc1_ref[...], 0.0)
  t = jnp.maximum(
      jnp.dot(t, wc2_ref[...], preferred_element_type=jnp.float32)
      + bc2_ref[...], 0.0)
  logit = jnp.dot(t, wc3_ref[...], preferred_element_type=jnp.float32) \
      + bc3_ref[...]
  prob_ref[...] = jax.nn.sigmoid(logit)


def _tc2(p, c, h, wl, wr, b, wc1, bc1, wc2, bc2, wc3, bc3):
  return pl.pallas_call(
      _tc2_body,
      out_shape=[
          jax.ShapeDtypeStruct((N, D), jnp.float32),
          jax.ShapeDtypeStruct((N, 1), jnp.float32),
      ],
  )(p, c, h, wl, wr, b, wc1, bc1, wc2, bc2, wc3, bc3)


def kernel(x, edge_index, Wl1, Wr1, b1, Wl2, Wr2, b2,
           Wc1, bc1, Wc2, bc2, Wc3, bc3):
  src = edge_index[0]
  dst = edge_index[1]
  z = jnp.zeros((ZR, D), jnp.float32)

  p1, cnt = _sc_agg_counts(x, src, dst, z)
  h = _tc1(p1, cnt, x, Wl1, Wr1, b1)
  (p2,) = _sc_agg(h, src, dst, z)
  emb, probs = _tc2(p2, cnt, h, Wl2, Wr2, b2, Wc1, bc1, Wc2, bc2, Wc3, bc3)
  return (emb, probs)
